# baseline (speedup 1.0000x reference)
import functools

import jax
import jax.numpy as jnp
from jax import lax
from jax.experimental import pallas as pl
from jax.experimental.pallas import tpu as pltpu
from jax.experimental.pallas import tpu_sc as plsc

_info = plsc.get_sparse_core_info()
_NC, _NS = _info.num_cores, _info.num_subcores
_NW = _NC * _NS

_D = 18432
_SPLIT = 4
_DQ = _D // _SPLIT
_ROWS = 4096
_ROWSQ = _ROWS * _SPLIT
_QB_PER_W = _ROWSQ // _NW
_CHUNK = 8
_N_CHUNKS = _QB_PER_W // _CHUNK


def _sc_gather(idx4, table_q):
    mesh = plsc.VectorSubcoreMesh(core_axis_name="c", subcore_axis_name="s")

    @functools.partial(
        pl.kernel,
        mesh=mesh,
        out_type=jax.ShapeDtypeStruct((_ROWSQ, _DQ), jnp.float32),
        scratch_types=[
            pltpu.VMEM((_QB_PER_W,), jnp.int32),
            pltpu.VMEM((_CHUNK, _DQ), jnp.float32),
            pltpu.SemaphoreType.DMA,
        ],
    )
    def k(idx_hbm, table_hbm, out_hbm, idx_v, buf, gsem):
        wid = lax.axis_index("s") * _NC + lax.axis_index("c")
        base = wid * _QB_PER_W
        pltpu.sync_copy(idx_hbm.at[pl.ds(base, _QB_PER_W)], idx_v)

        def body(g, carry):
            q0 = g * _CHUNK
            pltpu.async_copy(
                table_hbm.at[idx_v.at[pl.ds(q0, _CHUNK)]], buf, gsem
            ).wait()
            pltpu.sync_copy(buf, out_hbm.at[pl.ds(base + q0, _CHUNK)])
            return carry

        lax.fori_loop(0, _N_CHUNKS, body, 0)

    return k(idx4, table_q)


def kernel(prefix, table):
    idx = prefix.reshape(-1).astype(jnp.int32)
    idx4 = (idx[:, None] * _SPLIT + jnp.arange(_SPLIT, dtype=jnp.int32)).reshape(-1)
    table_q = table.reshape(_ROWS * 0 + table.shape[0] * _SPLIT, _DQ)
    out = _sc_gather(idx4, table_q)
    return out.reshape(prefix.shape[0], prefix.shape[1], table.shape[1])


# speedup vs baseline: 1.0523x; 1.0523x over previous
import functools

import jax
import jax.numpy as jnp
from jax import lax
from jax.experimental import pallas as pl
from jax.experimental.pallas import tpu as pltpu
from jax.experimental.pallas import tpu_sc as plsc

_info = plsc.get_sparse_core_info()
_NC, _NS = _info.num_cores, _info.num_subcores
_NW = _NC * _NS

_D = 18432
_SPLIT = 4
_DQ = _D // _SPLIT
_ROWS = 4096
_ROWSQ = _ROWS * _SPLIT
_QB_PER_W = _ROWSQ // _NW
_CHUNK = 8
_N_CHUNKS = _QB_PER_W // _CHUNK


def _sc_gather(idx4, table_q):
    mesh = plsc.VectorSubcoreMesh(core_axis_name="c", subcore_axis_name="s")

    @functools.partial(
        pl.kernel,
        mesh=mesh,
        out_type=jax.ShapeDtypeStruct((_ROWSQ, _DQ), jnp.float32),
        scratch_types=[
            pltpu.VMEM((_QB_PER_W,), jnp.int32),
            pltpu.VMEM((_CHUNK, _DQ), jnp.float32),
            pltpu.VMEM((_CHUNK, _DQ), jnp.float32),
            pltpu.SemaphoreType.DMA,
            pltpu.SemaphoreType.DMA,
            pltpu.SemaphoreType.DMA,
            pltpu.SemaphoreType.DMA,
        ],
    )
    def k(idx_hbm, table_hbm, out_hbm, idx_v, buf0, buf1, gs0, gs1, ss0, ss1):
        wid = lax.axis_index("s") * _NC + lax.axis_index("c")
        base = wid * _QB_PER_W
        pltpu.sync_copy(idx_hbm.at[pl.ds(base, _QB_PER_W)], idx_v)

        def gather_src(g):
            return table_hbm.at[idx_v.at[pl.ds(g * _CHUNK, _CHUNK)]]

        def out_dst(g):
            return out_hbm.at[pl.ds(base + g * _CHUNK, _CHUNK)]

        pltpu.async_copy(gather_src(0), buf0, gs0)

        def substep(g, cur, nxt, gs_cur, gs_nxt, ss_nxt):
            @pl.when(g > 0)
            def _():
                pltpu.make_async_copy(nxt, out_dst(g - 1), ss_nxt).wait()

            @pl.when(g + 1 < _N_CHUNKS)
            def _():
                pltpu.async_copy(gather_src(g + 1), nxt, gs_nxt)

            pltpu.make_async_copy(gather_src(g), cur, gs_cur).wait()

        def step(t, carry):
            g0 = t * 2
            substep(g0, buf0, buf1, gs0, gs1, ss1)
            pltpu.async_copy(buf0, out_dst(g0), ss0)
            substep(g0 + 1, buf1, buf0, gs1, gs0, ss0)
            pltpu.async_copy(buf1, out_dst(g0 + 1), ss1)
            return carry

        lax.fori_loop(0, _N_CHUNKS // 2, step, 0)
        pltpu.make_async_copy(buf1, out_dst(_N_CHUNKS - 1), ss1).wait()

    return k(idx4, table_q)


def kernel(prefix, table):
    idx = prefix.reshape(-1).astype(jnp.int32)
    idx4 = (idx[:, None] * _SPLIT + jnp.arange(_SPLIT, dtype=jnp.int32)).reshape(-1)
    table_q = table.reshape(table.shape[0] * _SPLIT, _DQ)
    out = _sc_gather(idx4, table_q)
    return out.reshape(prefix.shape[0], prefix.shape[1], table.shape[1])


# speedup vs baseline: 3.5434x; 3.3673x over previous
import functools

import jax
import jax.numpy as jnp
from jax import lax
from jax.experimental import pallas as pl
from jax.experimental.pallas import tpu as pltpu
from jax.experimental.pallas import tpu_sc as plsc

_info = plsc.get_sparse_core_info()
_NC, _NS = _info.num_cores, _info.num_subcores
_NW = _NC * _NS

_D = 18432
_SPLIT = 4
_DQ = _D // _SPLIT
_ROWS = 4096
_B_PER_W = _ROWS // _NW
_RCHUNK = 8
_N_RCHUNKS = _B_PER_W // _RCHUNK


def _sc_gather(idx, table):
    mesh = plsc.VectorSubcoreMesh(core_axis_name="c", subcore_axis_name="s")

    @functools.partial(
        pl.kernel,
        mesh=mesh,
        out_type=jax.ShapeDtypeStruct((_ROWS, _D), jnp.float32),
        scratch_types=[
            pltpu.VMEM((_B_PER_W,), jnp.int32),
            pltpu.VMEM((_RCHUNK, _DQ), jnp.float32),
            pltpu.VMEM((_RCHUNK, _DQ), jnp.float32),
            pltpu.SemaphoreType.DMA,
            pltpu.SemaphoreType.DMA,
            pltpu.SemaphoreType.DMA,
            pltpu.SemaphoreType.DMA,
        ],
    )
    def k(idx_hbm, table_hbm, out_hbm, idx_v, buf0, buf1, gs0, gs1, ss0, ss1):
        wid = lax.axis_index("s") * _NC + lax.axis_index("c")
        base = wid * _B_PER_W
        pltpu.sync_copy(idx_hbm.at[pl.ds(base, _B_PER_W)], idx_v)

        def unit_src(c, q):
            return table_hbm.at[
                idx_v.at[pl.ds(c * _RCHUNK, _RCHUNK)], pl.ds(q * _DQ, _DQ)
            ]

        def unit_dst(c, q):
            return out_hbm.at[
                pl.ds(base + c * _RCHUNK, _RCHUNK), pl.ds(q * _DQ, _DQ)
            ]

        pltpu.async_copy(unit_src(0, 0), buf0, gs0)

        def step(c, carry):
            for q in range(_SPLIT):
                cur, nxt = (buf0, buf1) if q % 2 == 0 else (buf1, buf0)
                gs_cur, gs_nxt = (gs0, gs1) if q % 2 == 0 else (gs1, gs0)
                ss_cur, ss_nxt = (ss0, ss1) if q % 2 == 0 else (ss1, ss0)
                if q == 0:
                    @pl.when(c > 0)
                    def _():
                        pltpu.make_async_copy(
                            nxt, unit_dst(c - 1, _SPLIT - 1), ss_nxt
                        ).wait()
                        pltpu.async_copy(unit_src(c, 1), nxt, gs_nxt)

                    @pl.when(c == 0)
                    def _():
                        pltpu.async_copy(unit_src(c, 1), nxt, gs_nxt)
                elif q < _SPLIT - 1:
                    pltpu.make_async_copy(nxt, unit_dst(c, q - 1), ss_nxt).wait()
                    pltpu.async_copy(unit_src(c, q + 1), nxt, gs_nxt)
                else:
                    pltpu.make_async_copy(nxt, unit_dst(c, q - 1), ss_nxt).wait()

                    @pl.when(c + 1 < _N_RCHUNKS)
                    def _():
                        pltpu.async_copy(unit_src(c + 1, 0), nxt, gs_nxt)

                pltpu.make_async_copy(unit_src(c, q), cur, gs_cur).wait()
                pltpu.async_copy(cur, unit_dst(c, q), ss_cur)
            return carry

        lax.fori_loop(0, _N_RCHUNKS, step, 0)
        pltpu.make_async_copy(
            buf1, unit_dst(_N_RCHUNKS - 1, _SPLIT - 1), ss1
        ).wait()

    return k(idx, table)


def kernel(prefix, table):
    idx = prefix.reshape(-1).astype(jnp.int32)
    out = _sc_gather(idx, table)
    return out.reshape(prefix.shape[0], prefix.shape[1], table.shape[1])
